# baseline (speedup 1.0000x reference)
import jax
import jax.numpy as jnp
from jax.experimental import pallas as pl
from jax.experimental.pallas import tpu as pltpu

K, D = 1024, 256
BM = 2048


def _vq_kernel(x_ref, et_ref, out_ref):
    x = x_ref[...]
    et = et_ref[...]
    codebook_sqr = jnp.sum(et * et, axis=0, keepdims=True)
    inputs_sqr = jnp.sum(x * x, axis=1, keepdims=True)
    mm = jnp.dot(x, et, preferred_element_type=jnp.float32)
    distances = (codebook_sqr + inputs_sqr) - 2.0 * mm
    dmin = jnp.min(distances, axis=1, keepdims=True)
    lane = jax.lax.broadcasted_iota(jnp.int32, (BM, K), 1)
    idx = jnp.min(jnp.where(distances == dmin, lane, K),
                  axis=1, keepdims=True)
    out_ref[...] = idx


def kernel(z_e_x, embedding):
    B, T, d = z_e_x.shape
    M = B * T
    x = z_e_x.reshape(M, d)
    et = embedding.T
    out = pl.pallas_call(
        _vq_kernel,
        grid=(M // BM,),
        in_specs=[
            pl.BlockSpec((BM, D), lambda i: (i, 0)),
            pl.BlockSpec((D, K), lambda i: (0, 0)),
        ],
        out_specs=pl.BlockSpec((BM, 1), lambda i: (i, 0)),
        out_shape=jax.ShapeDtypeStruct((M, 1), jnp.int32),
        compiler_params=pltpu.CompilerParams(
            dimension_semantics=("parallel",)),
    )(x, et)
    return out.reshape(B, T)


# speedup vs baseline: 1.1047x; 1.1047x over previous
import jax
import jax.numpy as jnp
from jax.experimental import pallas as pl
from jax.experimental.pallas import tpu as pltpu

K, D = 1024, 256
BM = 2048


def _vq_kernel(x_ref, e_ref, out_ref, et_ref, csq_ref):
    @pl.when(pl.program_id(0) == 0)
    def _init():
        et = e_ref[...].T
        et_ref[...] = et
        csq_ref[...] = jnp.sum(et * et, axis=0, keepdims=True)

    x = x_ref[...]
    et = et_ref[...]
    codebook_sqr = csq_ref[...]
    inputs_sqr = jnp.sum(x * x, axis=1, keepdims=True)
    mm = jnp.dot(x, et, preferred_element_type=jnp.float32)
    distances = (codebook_sqr + inputs_sqr) - 2.0 * mm
    dmin = jnp.min(distances, axis=1, keepdims=True)
    lane = jax.lax.broadcasted_iota(jnp.int32, (BM, K), 1)
    idx = jnp.min(jnp.where(distances == dmin, lane, K),
                  axis=1, keepdims=True)
    idx_row = jax.lax.transpose(idx, (1, 0))
    out_ref[...] = idx_row.reshape(1, 1, BM)


def kernel(z_e_x, embedding):
    B, T, d = z_e_x.shape
    M = B * T
    x = z_e_x.reshape(M, d)
    nblk = M // BM
    out = pl.pallas_call(
        _vq_kernel,
        grid=(nblk,),
        in_specs=[
            pl.BlockSpec((BM, D), lambda i: (i, 0)),
            pl.BlockSpec((K, D), lambda i: (0, 0)),
        ],
        out_specs=pl.BlockSpec((1, 1, BM), lambda i: (i, 0, 0)),
        out_shape=jax.ShapeDtypeStruct((nblk, 1, BM), jnp.int32),
        scratch_shapes=[
            pltpu.VMEM((D, K), jnp.float32),
            pltpu.VMEM((1, K), jnp.float32),
        ],
        compiler_params=pltpu.CompilerParams(
            dimension_semantics=("arbitrary",)),
    )(x, embedding)
    return out.reshape(B, T)


# speedup vs baseline: 1.3122x; 1.1879x over previous
import jax
import jax.numpy as jnp
from jax.experimental import pallas as pl
from jax.experimental.pallas import tpu as pltpu

K, D = 1024, 256
BM = 2048


def _vq_kernel(x_ref, e_ref, out_ref, et_ref, csq_ref):
    @pl.when(pl.program_id(0) == 0)
    def _init():
        et = e_ref[...].T
        et_ref[...] = et
        csq_ref[...] = jnp.sum(et * et, axis=0, keepdims=True)

    x = x_ref[...]
    et = et_ref[...]
    codebook_sqr = csq_ref[...]
    inputs_sqr = jnp.sum(x * x, axis=1, keepdims=True)
    mm = jnp.dot(x, et, preferred_element_type=jnp.float32)
    distances = (codebook_sqr + inputs_sqr) - 2.0 * mm
    dmin = jnp.min(distances, axis=1, keepdims=True)
    lane = jax.lax.broadcasted_iota(jnp.int32, (1, K), 1).astype(jnp.float32)
    idxf = jnp.min(jnp.where(distances == dmin, lane, float(K)),
                   axis=1, keepdims=True)
    idx_row = jax.lax.transpose(idxf, (1, 0))
    out_ref[...] = idx_row.astype(jnp.int32).reshape(1, 1, BM)


def kernel(z_e_x, embedding):
    B, T, d = z_e_x.shape
    M = B * T
    x = z_e_x.reshape(M, d)
    nblk = M // BM
    out = pl.pallas_call(
        _vq_kernel,
        grid=(nblk,),
        in_specs=[
            pl.BlockSpec((BM, D), lambda i: (i, 0)),
            pl.BlockSpec((K, D), lambda i: (0, 0)),
        ],
        out_specs=pl.BlockSpec((1, 1, BM), lambda i: (i, 0, 0)),
        out_shape=jax.ShapeDtypeStruct((nblk, 1, BM), jnp.int32),
        scratch_shapes=[
            pltpu.VMEM((D, K), jnp.float32),
            pltpu.VMEM((1, K), jnp.float32),
        ],
        compiler_params=pltpu.CompilerParams(
            dimension_semantics=("arbitrary",)),
    )(x, embedding)
    return out.reshape(B, T)


# speedup vs baseline: 1.3765x; 1.0489x over previous
import jax
import jax.numpy as jnp
from jax.experimental import pallas as pl
from jax.experimental.pallas import tpu as pltpu

K, D = 1024, 256
BM = 8192


def _vq_kernel(x_ref, e_ref, out_ref, et_ref, csq_ref):
    @pl.when(pl.program_id(0) == 0)
    def _init():
        et = e_ref[...].T
        et_ref[...] = et
        csq_ref[...] = jnp.sum(et * et, axis=0, keepdims=True)

    x = x_ref[...]
    et = et_ref[...]
    codebook_sqr = csq_ref[...]
    inputs_sqr = jnp.sum(x * x, axis=1, keepdims=True)
    mm = jnp.dot(x, et, preferred_element_type=jnp.float32)
    distances = (codebook_sqr + inputs_sqr) - 2.0 * mm
    dmin = jnp.min(distances, axis=1, keepdims=True)
    lane = jax.lax.broadcasted_iota(jnp.int32, (1, K), 1).astype(jnp.float32)
    idxf = jnp.min(jnp.where(distances == dmin, lane, float(K)),
                   axis=1, keepdims=True)
    idx_row = jax.lax.transpose(idxf, (1, 0))
    out_ref[...] = idx_row.astype(jnp.int32).reshape(1, 1, BM)


def kernel(z_e_x, embedding):
    B, T, d = z_e_x.shape
    M = B * T
    x = z_e_x.reshape(M, d)
    nblk = M // BM
    out = pl.pallas_call(
        _vq_kernel,
        grid=(nblk,),
        in_specs=[
            pl.BlockSpec((BM, D), lambda i: (i, 0)),
            pl.BlockSpec((K, D), lambda i: (0, 0)),
        ],
        out_specs=pl.BlockSpec((1, 1, BM), lambda i: (i, 0, 0)),
        out_shape=jax.ShapeDtypeStruct((nblk, 1, BM), jnp.int32),
        scratch_shapes=[
            pltpu.VMEM((D, K), jnp.float32),
            pltpu.VMEM((1, K), jnp.float32),
        ],
        compiler_params=pltpu.CompilerParams(
            dimension_semantics=("arbitrary",)),
    )(x, embedding)
    return out.reshape(B, T)


# speedup vs baseline: 1.3870x; 1.0076x over previous
import jax
import jax.numpy as jnp
from jax.experimental import pallas as pl
from jax.experimental.pallas import tpu as pltpu

K, D = 1024, 256
BM = 4096


def _vq_kernel(x_ref, e_ref, out_ref, et_ref, csq_ref):
    @pl.when(pl.program_id(0) == 0)
    def _init():
        et = e_ref[...].T
        et_ref[...] = et
        csq_ref[...] = jnp.sum(et * et, axis=0, keepdims=True)

    x = x_ref[...]
    et = et_ref[...]
    codebook_sqr = csq_ref[...]
    inputs_sqr = jnp.sum(x * x, axis=1, keepdims=True)
    mm = jnp.dot(x, et, preferred_element_type=jnp.float32)
    distances = (codebook_sqr + inputs_sqr) - 2.0 * mm
    dmin = jnp.min(distances, axis=1, keepdims=True)
    lane = jax.lax.broadcasted_iota(jnp.int32, (1, K), 1).astype(jnp.float32)
    idxf = jnp.min(jnp.where(distances == dmin, lane, float(K)),
                   axis=1, keepdims=True)
    idx_row = jax.lax.transpose(idxf, (1, 0))
    out_ref[...] = idx_row.astype(jnp.int32).reshape(1, 1, BM)


def kernel(z_e_x, embedding):
    B, T, d = z_e_x.shape
    M = B * T
    x = z_e_x.reshape(M, d)
    nblk = M // BM
    out = pl.pallas_call(
        _vq_kernel,
        grid=(nblk,),
        in_specs=[
            pl.BlockSpec((BM, D), lambda i: (i, 0)),
            pl.BlockSpec((K, D), lambda i: (0, 0)),
        ],
        out_specs=pl.BlockSpec((1, 1, BM), lambda i: (i, 0, 0)),
        out_shape=jax.ShapeDtypeStruct((nblk, 1, BM), jnp.int32),
        scratch_shapes=[
            pltpu.VMEM((D, K), jnp.float32),
            pltpu.VMEM((1, K), jnp.float32),
        ],
        compiler_params=pltpu.CompilerParams(
            dimension_semantics=("arbitrary",)),
    )(x, embedding)
    return out.reshape(B, T)
